# baseline (speedup 1.0000x reference)
import numpy as np
import jax
import jax.numpy as jnp
from jax.experimental import pallas as pl
from jax.experimental.pallas import tpu as pltpu

_U32 = np.uint32
_ROT1 = (13, 15, 26, 6)
_ROT2 = (17, 29, 16, 24)


def _threefry2x32_scalar(k0, k1, x0, x1):
    with np.errstate(over="ignore"):
        k0, k1 = _U32(k0), _U32(k1)
        ks = (k0, k1, _U32(k0 ^ k1 ^ _U32(0x1BD11BDA)))

        def rotl(v, d):
            return _U32((_U32(v) << _U32(d)) | (_U32(v) >> _U32(32 - d)))

        def four(x0, x1, rots):
            for r in rots:
                x0 = _U32(x0 + x1)
                x1 = _U32(x0 ^ rotl(x1, r))
            return x0, x1

        x0, x1 = _U32(x0 + ks[0]), _U32(x1 + ks[1])
        x0, x1 = four(x0, x1, _ROT1)
        x0, x1 = _U32(x0 + ks[1]), _U32(x1 + ks[2] + _U32(1))
        x0, x1 = four(x0, x1, _ROT2)
        x0, x1 = _U32(x0 + ks[2]), _U32(x1 + ks[0] + _U32(2))
        x0, x1 = four(x0, x1, _ROT1)
        x0, x1 = _U32(x0 + ks[0]), _U32(x1 + ks[1] + _U32(3))
        x0, x1 = four(x0, x1, _ROT2)
        x0, x1 = _U32(x0 + ks[1]), _U32(x1 + ks[2] + _U32(4))
        x0, x1 = four(x0, x1, _ROT1)
        return _U32(x0 + ks[2]), _U32(x1 + ks[0] + _U32(5))


def _subkey(key, j):
    y0, y1 = _threefry2x32_scalar(key[0], key[1], _U32(0), _U32(j))
    return (int(y0), int(y1))


_ROOT = (0, 42)
_K_MASK = _subkey(_ROOT, 0)
_K_FLIP = _subkey(_ROOT, 1)
_K_LO = _subkey(_K_FLIP, 1)

_N_CATEGORIES = 100000
_MASK_THRESH = 429496832

_ROWS = 3328
_LANES = 1024
_BLOCK_ROWS = 128


def _xor_bits(k, x1):
    ks0 = jnp.uint32(k[0])
    ks1 = jnp.uint32(k[1])
    ks2 = jnp.uint32(k[0] ^ k[1] ^ 0x1BD11BDA)

    def rotl(v, d):
        return (v << jnp.uint32(d)) | (v >> jnp.uint32(32 - d))

    def four(x0, x1, rots):
        for r in rots:
            x0 = x0 + x1
            x1 = x0 ^ rotl(x1, r)
        return x0, x1

    x0 = ks0
    x1 = x1 + ks1
    x0, x1 = four(x0, x1, _ROT1)
    x0, x1 = x0 + ks1, x1 + (ks2 + jnp.uint32(1))
    x0, x1 = four(x0, x1, _ROT2)
    x0, x1 = x0 + ks2, x1 + (ks0 + jnp.uint32(2))
    x0, x1 = four(x0, x1, _ROT1)
    x0, x1 = x0 + ks0, x1 + (ks1 + jnp.uint32(3))
    x0, x1 = four(x0, x1, _ROT2)
    x0, x1 = x0 + ks1, x1 + (ks2 + jnp.uint32(4))
    x0, x1 = four(x0, x1, _ROT1)
    return (x0 + ks2) ^ (x1 + (ks0 + jnp.uint32(5)))


def _sample_kernel(v_ref):
    c = pl.program_id(0)
    shape = (_BLOCK_ROWS, _LANES)
    row = jax.lax.broadcasted_iota(jnp.uint32, shape, 0)
    lane = jax.lax.broadcasted_iota(jnp.uint32, shape, 1)
    i = (jnp.uint32(c) * jnp.uint32(_BLOCK_ROWS) + row) * jnp.uint32(_LANES) + lane

    mbits = _xor_bits(_K_MASK, i)
    lobits = _xor_bits(_K_LO, i)
    keep = mbits < jnp.uint32(_MASK_THRESH)
    flips = (lobits % jnp.uint32(_N_CATEGORIES)).astype(jnp.int32)
    v_ref[...] = jnp.where(keep, jnp.int32(-1), flips)


def kernel(X):
    grid = (_ROWS // _BLOCK_ROWS,)
    v = pl.pallas_call(
        _sample_kernel,
        grid=grid,
        out_specs=pl.BlockSpec((_BLOCK_ROWS, _LANES), lambda c: (c, 0)),
        out_shape=jax.ShapeDtypeStruct((_ROWS, _LANES), jnp.int32),
        compiler_params=pltpu.CompilerParams(
            dimension_semantics=("arbitrary",),
        ),
    )()
    v3 = jnp.reshape(v, (8, 16384, 26))
    X_flips = jnp.where(v3 < 0, X, v3)
    X_pert = jnp.concatenate([X, X_flips], axis=0)
    return (X_pert, jnp.float32(0.0))


# speedup vs baseline: 1.2359x; 1.2359x over previous
import numpy as np
import jax
import jax.numpy as jnp
from jax.experimental import pallas as pl
from jax.experimental.pallas import tpu as pltpu

_U32 = np.uint32
_ROT1 = (13, 15, 26, 6)
_ROT2 = (17, 29, 16, 24)


def _threefry2x32_scalar(k0, k1, x0, x1):
    with np.errstate(over="ignore"):
        k0, k1 = _U32(k0), _U32(k1)
        ks = (k0, k1, _U32(k0 ^ k1 ^ _U32(0x1BD11BDA)))

        def rotl(v, d):
            return _U32((_U32(v) << _U32(d)) | (_U32(v) >> _U32(32 - d)))

        def four(x0, x1, rots):
            for r in rots:
                x0 = _U32(x0 + x1)
                x1 = _U32(x0 ^ rotl(x1, r))
            return x0, x1

        x0, x1 = _U32(x0 + ks[0]), _U32(x1 + ks[1])
        x0, x1 = four(x0, x1, _ROT1)
        x0, x1 = _U32(x0 + ks[1]), _U32(x1 + ks[2] + _U32(1))
        x0, x1 = four(x0, x1, _ROT2)
        x0, x1 = _U32(x0 + ks[2]), _U32(x1 + ks[0] + _U32(2))
        x0, x1 = four(x0, x1, _ROT1)
        x0, x1 = _U32(x0 + ks[0]), _U32(x1 + ks[1] + _U32(3))
        x0, x1 = four(x0, x1, _ROT2)
        x0, x1 = _U32(x0 + ks[1]), _U32(x1 + ks[2] + _U32(4))
        x0, x1 = four(x0, x1, _ROT1)
        return _U32(x0 + ks[2]), _U32(x1 + ks[0] + _U32(5))


def _subkey(key, j):
    y0, y1 = _threefry2x32_scalar(key[0], key[1], _U32(0), _U32(j))
    return (int(y0), int(y1))


_ROOT = (0, 42)
_K_MASK = _subkey(_ROOT, 0)
_K_FLIP = _subkey(_ROOT, 1)
_K_LO = _subkey(_K_FLIP, 1)

_N_CATEGORIES = 100000
_MASK_THRESH = 429496832

_B = 8
_ROWS = 16384
_C = 26
_S = 4096


def _xor_bits(k, x1):
    ks0 = jnp.uint32(k[0])
    ks1 = jnp.uint32(k[1])
    ks2 = jnp.uint32(k[0] ^ k[1] ^ 0x1BD11BDA)

    def rotl(v, d):
        return (v << jnp.uint32(d)) | (v >> jnp.uint32(32 - d))

    def four(x0, x1, rots):
        for r in rots:
            x0 = x0 + x1
            x1 = x0 ^ rotl(x1, r)
        return x0, x1

    x0 = ks0
    x1 = x1 + ks1
    x0, x1 = four(x0, x1, _ROT1)
    x0, x1 = x0 + ks1, x1 + (ks2 + jnp.uint32(1))
    x0, x1 = four(x0, x1, _ROT2)
    x0, x1 = x0 + ks2, x1 + (ks0 + jnp.uint32(2))
    x0, x1 = four(x0, x1, _ROT1)
    x0, x1 = x0 + ks0, x1 + (ks1 + jnp.uint32(3))
    x0, x1 = four(x0, x1, _ROT2)
    x0, x1 = x0 + ks1, x1 + (ks2 + jnp.uint32(4))
    x0, x1 = four(x0, x1, _ROT1)
    return (x0 + ks2) ^ (x1 + (ks0 + jnp.uint32(5)))


def _sample_kernel(v_ref):
    b = pl.program_id(0)
    cs = pl.program_id(1)

    shape_t = (_C, _S)
    row = jax.lax.broadcasted_iota(jnp.uint32, shape_t, 0)
    col = jax.lax.broadcasted_iota(jnp.uint32, shape_t, 1)
    base = (jnp.uint32(b) * jnp.uint32(_ROWS) +
            jnp.uint32(cs) * jnp.uint32(_S)) * jnp.uint32(_C)
    i = base + col * jnp.uint32(_C) + row

    mbits = _xor_bits(_K_MASK, i)
    lobits = _xor_bits(_K_LO, i)
    keep = mbits < jnp.uint32(_MASK_THRESH)
    flips = (lobits % jnp.uint32(_N_CATEGORIES)).astype(jnp.int32)
    v_t = jnp.where(keep, jnp.int32(-1), flips)
    v_ref[...] = jnp.swapaxes(v_t, 0, 1)[None]


def kernel(X):
    grid = (_B, _ROWS // _S)
    v = pl.pallas_call(
        _sample_kernel,
        grid=grid,
        out_specs=pl.BlockSpec((1, _S, _C), lambda b, cs: (b, cs, 0)),
        out_shape=jax.ShapeDtypeStruct((_B, _ROWS, _C), jnp.int32),
        compiler_params=pltpu.CompilerParams(
            dimension_semantics=("arbitrary", "arbitrary"),
        ),
    )()
    X_flips = jnp.where(v < 0, X, v)
    X_pert = jnp.concatenate([X, X_flips], axis=0)
    return (X_pert, jnp.float32(0.0))


# speedup vs baseline: 1.4599x; 1.1812x over previous
import numpy as np
import jax
import jax.numpy as jnp
from jax.experimental import pallas as pl
from jax.experimental.pallas import tpu as pltpu

_U32 = np.uint32
_ROT1 = (13, 15, 26, 6)
_ROT2 = (17, 29, 16, 24)


def _threefry2x32_scalar(k0, k1, x0, x1):
    with np.errstate(over="ignore"):
        k0, k1 = _U32(k0), _U32(k1)
        ks = (k0, k1, _U32(k0 ^ k1 ^ _U32(0x1BD11BDA)))

        def rotl(v, d):
            return _U32((_U32(v) << _U32(d)) | (_U32(v) >> _U32(32 - d)))

        def four(x0, x1, rots):
            for r in rots:
                x0 = _U32(x0 + x1)
                x1 = _U32(x0 ^ rotl(x1, r))
            return x0, x1

        x0, x1 = _U32(x0 + ks[0]), _U32(x1 + ks[1])
        x0, x1 = four(x0, x1, _ROT1)
        x0, x1 = _U32(x0 + ks[1]), _U32(x1 + ks[2] + _U32(1))
        x0, x1 = four(x0, x1, _ROT2)
        x0, x1 = _U32(x0 + ks[2]), _U32(x1 + ks[0] + _U32(2))
        x0, x1 = four(x0, x1, _ROT1)
        x0, x1 = _U32(x0 + ks[0]), _U32(x1 + ks[1] + _U32(3))
        x0, x1 = four(x0, x1, _ROT2)
        x0, x1 = _U32(x0 + ks[1]), _U32(x1 + ks[2] + _U32(4))
        x0, x1 = four(x0, x1, _ROT1)
        return _U32(x0 + ks[2]), _U32(x1 + ks[0] + _U32(5))


def _subkey(key, j):
    y0, y1 = _threefry2x32_scalar(key[0], key[1], _U32(0), _U32(j))
    return (int(y0), int(y1))


_ROOT = (0, 42)
_K_MASK = _subkey(_ROOT, 0)
_K_FLIP = _subkey(_ROOT, 1)
_K_LO = _subkey(_K_FLIP, 1)

_N_CATEGORIES = 100000
_MASK_THRESH = 429496832

_B = 8
_ROWS = 16384
_C = 26
_L = 4096


def _xor_bits(k, x1):
    ks0 = jnp.uint32(k[0])
    ks1 = jnp.uint32(k[1])
    ks2 = jnp.uint32(k[0] ^ k[1] ^ 0x1BD11BDA)

    def rotl(v, d):
        return (v << jnp.uint32(d)) | (v >> jnp.uint32(32 - d))

    def four(x0, x1, rots):
        for r in rots:
            x0 = x0 + x1
            x1 = x0 ^ rotl(x1, r)
        return x0, x1

    x0 = ks0
    x1 = x1 + ks1
    x0, x1 = four(x0, x1, _ROT1)
    x0, x1 = x0 + ks1, x1 + (ks2 + jnp.uint32(1))
    x0, x1 = four(x0, x1, _ROT2)
    x0, x1 = x0 + ks2, x1 + (ks0 + jnp.uint32(2))
    x0, x1 = four(x0, x1, _ROT1)
    x0, x1 = x0 + ks0, x1 + (ks1 + jnp.uint32(3))
    x0, x1 = four(x0, x1, _ROT2)
    x0, x1 = x0 + ks1, x1 + (ks2 + jnp.uint32(4))
    x0, x1 = four(x0, x1, _ROT1)
    return (x0 + ks2) ^ (x1 + (ks0 + jnp.uint32(5)))


def _perturb_kernel(xt_ref, fl_ref):
    b = pl.program_id(0)
    cs = pl.program_id(1)
    xt = xt_ref[0]

    shape_t = (_C, _L)
    row = jax.lax.broadcasted_iota(jnp.uint32, shape_t, 0)
    col = jax.lax.broadcasted_iota(jnp.uint32, shape_t, 1)
    base = (jnp.uint32(b) * jnp.uint32(_ROWS) +
            jnp.uint32(cs) * jnp.uint32(_L)) * jnp.uint32(_C)
    i = base + col * jnp.uint32(_C) + row

    mbits = _xor_bits(_K_MASK, i)
    lobits = _xor_bits(_K_LO, i)
    keep = mbits < jnp.uint32(_MASK_THRESH)
    flips = (lobits % jnp.uint32(_N_CATEGORIES)).astype(jnp.int32)
    fl_ref[0] = jnp.where(keep, xt, flips)


def kernel(X):
    Xt = jnp.swapaxes(X, 1, 2)
    grid = (_B, _ROWS // _L)
    fl_t = pl.pallas_call(
        _perturb_kernel,
        grid=grid,
        in_specs=[pl.BlockSpec((1, _C, _L), lambda b, cs: (b, 0, cs))],
        out_specs=pl.BlockSpec((1, _C, _L), lambda b, cs: (b, 0, cs)),
        out_shape=jax.ShapeDtypeStruct((_B, _C, _ROWS), jnp.int32),
        compiler_params=pltpu.CompilerParams(
            dimension_semantics=("arbitrary", "arbitrary"),
        ),
    )(Xt)
    X_flips = jnp.swapaxes(fl_t, 1, 2)
    X_pert = jnp.concatenate([X, X_flips], axis=0)
    return (X_pert, jnp.float32(0.0))


# speedup vs baseline: 1.5048x; 1.0307x over previous
import numpy as np
import jax
import jax.numpy as jnp
from jax.experimental import pallas as pl
from jax.experimental.pallas import tpu as pltpu

_U32 = np.uint32
_ROT1 = (13, 15, 26, 6)
_ROT2 = (17, 29, 16, 24)


def _threefry2x32_scalar(k0, k1, x0, x1):
    with np.errstate(over="ignore"):
        k0, k1 = _U32(k0), _U32(k1)
        ks = (k0, k1, _U32(k0 ^ k1 ^ _U32(0x1BD11BDA)))

        def rotl(v, d):
            return _U32((_U32(v) << _U32(d)) | (_U32(v) >> _U32(32 - d)))

        def four(x0, x1, rots):
            for r in rots:
                x0 = _U32(x0 + x1)
                x1 = _U32(x0 ^ rotl(x1, r))
            return x0, x1

        x0, x1 = _U32(x0 + ks[0]), _U32(x1 + ks[1])
        x0, x1 = four(x0, x1, _ROT1)
        x0, x1 = _U32(x0 + ks[1]), _U32(x1 + ks[2] + _U32(1))
        x0, x1 = four(x0, x1, _ROT2)
        x0, x1 = _U32(x0 + ks[2]), _U32(x1 + ks[0] + _U32(2))
        x0, x1 = four(x0, x1, _ROT1)
        x0, x1 = _U32(x0 + ks[0]), _U32(x1 + ks[1] + _U32(3))
        x0, x1 = four(x0, x1, _ROT2)
        x0, x1 = _U32(x0 + ks[1]), _U32(x1 + ks[2] + _U32(4))
        x0, x1 = four(x0, x1, _ROT1)
        return _U32(x0 + ks[2]), _U32(x1 + ks[0] + _U32(5))


def _subkey(key, j):
    y0, y1 = _threefry2x32_scalar(key[0], key[1], _U32(0), _U32(j))
    return (int(y0), int(y1))


_ROOT = (0, 42)
_K_MASK = _subkey(_ROOT, 0)
_K_FLIP = _subkey(_ROOT, 1)
_K_LO = _subkey(_K_FLIP, 1)

_N_CATEGORIES = 100000
_MASK_THRESH = 429496832

_TOTAL_ROWS = 131072
_C = 26
_S = 8192


def _xor_bits(k, x1):
    ks0 = jnp.uint32(k[0])
    ks1 = jnp.uint32(k[1])
    ks2 = jnp.uint32(k[0] ^ k[1] ^ 0x1BD11BDA)

    def rotl(v, d):
        return (v << jnp.uint32(d)) | (v >> jnp.uint32(32 - d))

    def four(x0, x1, rots):
        for r in rots:
            x0 = x0 + x1
            x1 = x0 ^ rotl(x1, r)
        return x0, x1

    x0 = ks0
    x1 = x1 + ks1
    x0, x1 = four(x0, x1, _ROT1)
    x0, x1 = x0 + ks1, x1 + (ks2 + jnp.uint32(1))
    x0, x1 = four(x0, x1, _ROT2)
    x0, x1 = x0 + ks2, x1 + (ks0 + jnp.uint32(2))
    x0, x1 = four(x0, x1, _ROT1)
    x0, x1 = x0 + ks0, x1 + (ks1 + jnp.uint32(3))
    x0, x1 = four(x0, x1, _ROT2)
    x0, x1 = x0 + ks1, x1 + (ks2 + jnp.uint32(4))
    x0, x1 = four(x0, x1, _ROT1)
    return (x0 + ks2) ^ (x1 + (ks0 + jnp.uint32(5)))


def _perturb_kernel(x_ref, out_ref):
    c = pl.program_id(0)
    x = x_ref[...]

    shape_t = (_C, _S)
    row = jax.lax.broadcasted_iota(jnp.uint32, shape_t, 0)
    col = jax.lax.broadcasted_iota(jnp.uint32, shape_t, 1)
    i = jnp.uint32(c) * jnp.uint32(_S * _C) + col * jnp.uint32(_C) + row

    mbits = _xor_bits(_K_MASK, i)
    lobits = _xor_bits(_K_LO, i)
    keep = mbits < jnp.uint32(_MASK_THRESH)
    flips = (lobits % jnp.uint32(_N_CATEGORIES)).astype(jnp.int32)
    v_t = jnp.where(keep, jnp.int32(-1), flips)
    v = jnp.swapaxes(v_t, 0, 1)

    out_ref[0] = x
    out_ref[1] = jnp.where(v < jnp.int32(0), x, v)


def kernel(X):
    x2 = jnp.reshape(X, (_TOTAL_ROWS, _C))
    grid = (_TOTAL_ROWS // _S,)
    out = pl.pallas_call(
        _perturb_kernel,
        grid=grid,
        in_specs=[pl.BlockSpec((_S, _C), lambda c: (c, 0))],
        out_specs=pl.BlockSpec((2, _S, _C), lambda c: (0, c, 0)),
        out_shape=jax.ShapeDtypeStruct((2, _TOTAL_ROWS, _C), jnp.int32),
        compiler_params=pltpu.CompilerParams(
            dimension_semantics=("arbitrary",),
        ),
    )(x2)
    X_pert = jnp.reshape(out, (16, 16384, 26))
    return (X_pert, jnp.float32(0.0))
